# baseline (speedup 1.0000x reference)
import functools

import jax
import jax.numpy as jnp
from jax import lax
from jax.experimental import pallas as pl
from jax.experimental.pallas import tpu as pltpu
from jax.experimental.pallas import tpu_sc as plsc


def _make_gather(num_ids_rows: int, num_ids_cols: int, dim: int):
    info = plsc.get_sparse_core_info()
    nc, ns = info.num_cores, info.num_subcores
    nw = nc * ns
    batch = num_ids_rows * num_ids_cols
    assert batch % (8 * nw) == 0
    b_per_w = batch // nw
    assert num_ids_cols % b_per_w == 0 or b_per_w % num_ids_cols == 0
    per_row = num_ids_cols // b_per_w
    mesh = plsc.VectorSubcoreMesh(core_axis_name="c", subcore_axis_name="s")

    @functools.partial(
        pl.kernel,
        mesh=mesh,
        compiler_params=pltpu.CompilerParams(
            use_tc_tiling_on_sc=False,
            disable_bounds_checks=True,
            disable_semaphore_checks=True,
            skip_device_barrier=True,
        ),
        out_type=jax.ShapeDtypeStruct((num_ids_rows, num_ids_cols, dim), jnp.float32),
        scratch_types=[
            pltpu.VMEM((b_per_w,), jnp.int32),
            pltpu.VMEM((b_per_w, dim), jnp.float32),
            pltpu.SemaphoreType.DMA,
        ],
    )
    def gather_kernel(table_hbm, idx_hbm, out_hbm, idx_v, rows_v, sem):
        wid = lax.axis_index("s") * nc + lax.axis_index("c")
        r = wid // per_row
        col = (wid % per_row) * b_per_w
        pltpu.sync_copy(idx_hbm.at[r, pl.ds(col, b_per_w)], idx_v)
        pltpu.async_copy(table_hbm.at[idx_v], rows_v, sem).wait()
        pltpu.sync_copy(rows_v, out_hbm.at[r, pl.ds(col, b_per_w)])

    return gather_kernel


def kernel(embed_id, weight):
    num_rows, dim = weight.shape
    ir, ic = embed_id.shape
    out = _make_gather(ir, ic, dim)(weight, embed_id.astype(jnp.int32))
    return out


# speedup vs baseline: 1.0878x; 1.0878x over previous
import functools

import jax
import jax.numpy as jnp
from jax import lax
from jax.experimental import pallas as pl
from jax.experimental.pallas import tpu as pltpu
from jax.experimental.pallas import tpu_sc as plsc

_LANES = 128


def _make_gather(num_rows: int, num_ids_rows: int, num_ids_cols: int, dim: int):
    info = plsc.get_sparse_core_info()
    nc, ns = info.num_cores, info.num_subcores
    nw = nc * ns
    batch = num_ids_rows * num_ids_cols
    b_per_w = batch // nw
    per_row = num_ids_cols // b_per_w
    mesh = plsc.VectorSubcoreMesh(core_axis_name="c", subcore_axis_name="s")

    chunk = 256
    n_chunks = b_per_w // chunk

    @functools.partial(
        pl.kernel,
        mesh=mesh,
        out_type=jax.ShapeDtypeStruct((num_ids_rows, num_ids_cols, dim), jnp.float32),
        scratch_types=[
            pltpu.VMEM((b_per_w,), jnp.int32),
            pltpu.VMEM((chunk, _LANES), jnp.float32),
            pltpu.VMEM((chunk, dim), jnp.float32),
            pltpu.SemaphoreType.DMA,
        ],
    )
    def gather_kernel(table_hbm, idx_hbm, out_hbm, idx_v, rows_w, rows_c, sem):
        wid = lax.axis_index("s") * nc + lax.axis_index("c")
        r = wid // per_row
        col = (wid % per_row) * b_per_w
        pltpu.sync_copy(idx_hbm.at[r, pl.ds(col, b_per_w)], idx_v)
        for c in range(n_chunks):
            pltpu.async_copy(
                table_hbm.at[idx_v.at[pl.ds(c * chunk, chunk)]], rows_w, sem
            ).wait()

            def compact_row(i, carry):
                for j in range(dim // 16):
                    rows_c[i, pl.ds(j * 16, 16)] = rows_w[i, pl.ds(j * 16, 16)]
                return carry

            lax.fori_loop(0, chunk, compact_row, 0)
            pltpu.sync_copy(rows_c, out_hbm.at[r, pl.ds(col + c * chunk, chunk)])

    return gather_kernel


def kernel(embed_id, weight):
    num_rows, dim = weight.shape
    ir, ic = embed_id.shape
    wpad = jnp.pad(weight, ((0, 0), (0, _LANES - dim)))
    out = _make_gather(num_rows, ir, ic, dim)(wpad, embed_id.astype(jnp.int32))
    return out
